# Initial kernel scaffold; baseline (speedup 1.0000x reference)
import jax
import jax.numpy as jnp
from jax.experimental import pallas as pl


def kernel(x, edge_index, W_pre, b_pre, Ws1, bs1, Ws2, bs2):
    raise NotImplementedError("write your pallas kernel here")


# speedup vs baseline: 6.2891x; 6.2891x over previous
import functools

import jax
import jax.numpy as jnp
from jax import lax
from jax.experimental import pallas as pl
from jax.experimental.pallas import tpu as pltpu
from jax.experimental.pallas import tpu_sc as plsc

_CHUNK = 128
_NC = 2
_NS = 16
_NW = _NC * _NS


def _copy_plan(total):
    plan = []
    off = 0
    while off < total:
        sz = min(_CHUNK, total - off)
        plan.append((off, sz))
        off += sz
    return plan


@functools.lru_cache(maxsize=None)
def _make_agg(n, e, d):
    assert e % _CHUNK == 0 and d % 16 == 0 and n % 8 == 0
    n_chunks = e // _CHUNK
    rounds = -(-n_chunks // _NW)
    base_rows = ((n + _NS - 1) // _NS + 7) // 8 * 8
    tail_rows = n - (_NS - 1) * base_rows
    assert 0 <= tail_rows <= base_rows and tail_rows % 8 == 0
    mesh = plsc.VectorSubcoreMesh(core_axis_name="c", subcore_axis_name="s")

    @functools.partial(
        pl.kernel,
        mesh=mesh,
        out_type=jax.ShapeDtypeStruct((_NC * n, d), jnp.float32),
        scratch_types=[
            pltpu.VMEM((_CHUNK,), jnp.int32),
            pltpu.VMEM((_CHUNK,), jnp.int32),
            pltpu.VMEM((_CHUNK, d), jnp.float32),
            pltpu.VMEM((_CHUNK, d), jnp.float32),
            pltpu.VMEM_SHARED((n, d), jnp.float32),
            pltpu.SemaphoreType.DMA,
        ],
    )
    def agg(h_hbm, src_hbm, dst_hbm, out_hbm, src_v, dst_v, rows_v, zbuf_v,
            acc_sh, sem):
        cid = lax.axis_index("c")
        sid = lax.axis_index("s")
        wid = sid * _NC + cid

        zeros16 = jnp.zeros((16,), jnp.float32)

        def zrow(r, carry):
            for cc in range(d // 16):
                zbuf_v[r, pl.ds(cc * 16, 16)] = zeros16
            return carry

        lax.fori_loop(0, _CHUNK, zrow, None)

        row0 = sid * base_rows

        def zero_slice(nrows):
            for off, sz in _copy_plan(nrows):
                pltpu.sync_copy(zbuf_v.at[pl.ds(0, sz)],
                                acc_sh.at[pl.ds(row0 + off, sz)])

        @pl.when(sid < _NS - 1)
        def _():
            zero_slice(base_rows)

        @pl.when(sid == _NS - 1)
        def _():
            zero_slice(tail_rows)

        plsc.subcore_barrier()

        def body(j, carry):
            chunk = wid + _NW * j

            @pl.when(chunk < n_chunks)
            def _():
                base = chunk * _CHUNK
                pltpu.sync_copy(src_hbm.at[pl.ds(base, _CHUNK)], src_v)
                pltpu.sync_copy(dst_hbm.at[pl.ds(base, _CHUNK)], dst_v)
                pltpu.async_copy(h_hbm.at[src_v], rows_v, sem).wait()
                pltpu.sync_copy(rows_v, acc_sh.at[dst_v], add=True)

            return carry

        lax.fori_loop(0, rounds, body, None)
        plsc.subcore_barrier()

        out_row0 = cid * n + row0

        def write_slice(nrows):
            for off, sz in _copy_plan(nrows):
                pltpu.sync_copy(acc_sh.at[pl.ds(row0 + off, sz)],
                                zbuf_v.at[pl.ds(0, sz)])
                pltpu.sync_copy(zbuf_v.at[pl.ds(0, sz)],
                                out_hbm.at[pl.ds(out_row0 + off, sz)])

        @pl.when(sid < _NS - 1)
        def _():
            write_slice(base_rows)

        @pl.when(sid == _NS - 1)
        def _():
            write_slice(tail_rows)

    return agg


def _linear_body(x_ref, w_ref, b_ref, o_ref):
    o_ref[...] = (jnp.dot(x_ref[...], w_ref[...],
                          preferred_element_type=jnp.float32) + b_ref[...])


def _mlp_body(h_ref, a0_ref, a1_ref, w1_ref, b1_ref, w2_ref, b2_ref, o_ref):
    z = h_ref[...] + a0_ref[...] + a1_ref[...]
    t = jnp.maximum(jnp.dot(z, w1_ref[...],
                            preferred_element_type=jnp.float32) + b1_ref[...],
                    0.0)
    t = jnp.dot(t, w2_ref[...], preferred_element_type=jnp.float32) + b2_ref[...]
    o_ref[...] = jnp.maximum(t, 0.0)


def _row_block(n):
    for blk in (2000, 1000, 500, 250, 125):
        if n % blk == 0:
            return blk
    return n


def _linear(x, w, b):
    n, _ = x.shape
    d = w.shape[1]
    blk = _row_block(n)
    return pl.pallas_call(
        _linear_body,
        grid=(n // blk,),
        in_specs=[
            pl.BlockSpec((blk, x.shape[1]), lambda i: (i, 0)),
            pl.BlockSpec((x.shape[1], d), lambda i: (0, 0)),
            pl.BlockSpec((1, d), lambda i: (0, 0)),
        ],
        out_specs=pl.BlockSpec((blk, d), lambda i: (i, 0)),
        out_shape=jax.ShapeDtypeStruct((n, d), jnp.float32),
    )(x, w, b.reshape(1, d))


def _mlp(h, parts, w1, b1, w2, b2):
    n, d = h.shape
    blk = _row_block(n)
    nb = n // blk
    return pl.pallas_call(
        _mlp_body,
        grid=(nb,),
        in_specs=[
            pl.BlockSpec((blk, d), lambda i: (i, 0)),
            pl.BlockSpec((blk, d), lambda i: (i, 0)),
            pl.BlockSpec((blk, d), lambda i, _nb=nb: (i + _nb, 0)),
            pl.BlockSpec((d, d), lambda i: (0, 0)),
            pl.BlockSpec((1, d), lambda i: (0, 0)),
            pl.BlockSpec((d, d), lambda i: (0, 0)),
            pl.BlockSpec((1, d), lambda i: (0, 0)),
        ],
        out_specs=pl.BlockSpec((blk, d), lambda i: (i, 0)),
        out_shape=jax.ShapeDtypeStruct((n, d), jnp.float32),
    )(h, parts, parts, w1, b1.reshape(1, d), w2, b2.reshape(1, d))


def kernel(x, edge_index, W_pre, b_pre, Ws1, bs1, Ws2, bs2):
    n = x.shape[0]
    d = W_pre.shape[1]
    e = edge_index.shape[1]
    layers = Ws1.shape[0]
    src = edge_index[0]
    dst = edge_index[1]

    agg_fn = _make_agg(n, e, d)
    h = _linear(x, W_pre, b_pre)
    for l in range(layers):
        parts = agg_fn(h, src, dst)
        h = _mlp(h, parts, Ws1[l], bs1[l], Ws2[l], bs2[l])
    return h
